# baseline (speedup 1.0000x reference)
import functools

import jax
import jax.numpy as jnp
import numpy as np
from jax import lax
from jax.experimental import pallas as pl
from jax.experimental.pallas import tpu as pltpu
from jax.experimental.pallas import tpu_sc as plsc

NT = 16
NC = 2
KB = 128
PAD_ROWS = 240


def _sc_aggregate(tbl, src_r, dst_r, B, N, H, NQ, NBLK):
    NP = N + PAD_ROWS
    ZR = NP // NT
    ZSUB = ZR // 2
    WS = (N // NT) & ~7
    WTAIL = N - NT * WS

    mesh = plsc.VectorSubcoreMesh(
        core_axis_name="c", subcore_axis_name="s",
        num_cores=NC, num_subcores=NT)

    @functools.partial(
        pl.kernel,
        out_type=jax.ShapeDtypeStruct((B * NQ, N, H), jnp.float32),
        mesh=mesh,
        scratch_types=[
            pltpu.VMEM((NBLK, KB), jnp.int32),
            pltpu.VMEM((NBLK, KB), jnp.int32),
            pltpu.VMEM((NBLK, KB), jnp.int32),
            pltpu.VMEM((4, KB, H), jnp.float32),
            pltpu.VMEM((ZSUB, H), jnp.float32),
            pltpu.VMEM_SHARED((NP, H), jnp.float32),
            [pltpu.SemaphoreType.DMA] * 4,
            [pltpu.SemaphoreType.DMA] * 4,
        ],
        compiler_params=pltpu.CompilerParams(use_tc_tiling_on_sc=False),
    )
    def agg_kernel(tbl_hbm, src_hbm, dst_hbm, out_hbm,
                   src_v, idx_v, dst_v, rows_v, zbuf_v, acc_sh, gsems, ssems):
        c = lax.axis_index("c")
        s = lax.axis_index("s")

        pltpu.sync_copy(src_hbm.at[s], src_v)
        pltpu.sync_copy(dst_hbm.at[s], dst_v)

        def fill_idx(q):
            def frow(i, _):
                for jj in range(KB // 16):
                    sl = pl.ds(jj * 16, 16)
                    idx_v[i, sl] = src_v[i, sl] * NQ + q
                return 0
            lax.fori_loop(0, NBLK, frow, 0)

        zvec = jnp.zeros((16,), jnp.float32)

        def zrow(i, _):
            for j in range(H // 16):
                zbuf_v[i, pl.ds(j * 16, 16)] = zvec
            return 0

        lax.fori_loop(0, ZSUB, zrow, 0)

        def chunk(b, q):
            tblk = tbl_hbm.at[b]
            outk = out_hbm.at[b * NQ + q]
            for r in range(2):
                pltpu.sync_copy(zbuf_v, acc_sh.at[pl.ds(s * ZR + r * ZSUB, ZSUB)])
            plsc.subcore_barrier()

            def issue_g(j, t):
                pltpu.async_copy(tblk.at[idx_v.at[j]], rows_v.at[t], gsems[t])

            def wait_g(j, t):
                pltpu.make_async_copy(
                    tblk.at[idx_v.at[j]], rows_v.at[t], gsems[t]).wait()

            def issue_s(j, t):
                pltpu.async_copy(rows_v.at[t], acc_sh.at[dst_v.at[j]],
                                 ssems[t], add=True)

            def wait_s(j, t):
                pltpu.make_async_copy(
                    rows_v.at[t], acc_sh.at[dst_v.at[j]], ssems[t]).wait()

            for t in range(4):
                issue_g(t, t)

            def step(g, _):
                j0 = 4 * g
                for t in range(4):
                    wait_g(j0 + t, t)
                    issue_s(j0 + t, t)
                for t in range(4):
                    wait_s(j0 + t, t)
                    issue_g(j0 + t + 4, t)
                return 0

            lax.fori_loop(0, NBLK // 4 - 1, step, 0)
            j0 = NBLK - 4
            for t in range(4):
                wait_g(j0 + t, t)
                issue_s(j0 + t, t)
            for t in range(4):
                wait_s(j0 + t, t)
            plsc.subcore_barrier()

            pltpu.sync_copy(acc_sh.at[pl.ds(s * WS, WS)],
                            outk.at[pl.ds(s * WS, WS)])
            if WTAIL:
                @pl.when(s == NT - 1)
                def _():
                    pltpu.sync_copy(acc_sh.at[pl.ds(NT * WS, WTAIL)],
                                    outk.at[pl.ds(NT * WS, WTAIL)])
            plsc.subcore_barrier()

        qpc = NQ // NC
        for half in range(NC):
            @pl.when(c == half)
            def _():
                for j in range(qpc):
                    q = half * qpc + j
                    fill_idx(q)
                    for b in range(B):
                        chunk(b, q)

    return agg_kernel(tbl, src_r, dst_r)


def _tc_epilogue(agg, node, W, bvec, gamma, beta, B, N, D, H, NQ, BLK):
    PB = BLK // 2

    def lnact(y, g, be):
        mu = jnp.mean(y, axis=-1, keepdims=True)
        yc = y - mu
        var = jnp.mean(yc * yc, axis=-1, keepdims=True)
        ln = yc * lax.rsqrt(var + 1e-5) * g + be
        return 0.5 * ln * (1.0 + lax.erf(ln * np.float32(1.0 / np.sqrt(2.0))))

    def body(agg_ref, node_ref, w_ref, b_ref, g_ref, be_ref, out_ref):
        ye = b_ref[...]
        yo = b_ref[...]
        for q in range(NQ):
            a = agg_ref[0, q]
            wq = w_ref[q * H:(q + 1) * H, :]
            ye = ye + jnp.dot(a[:, :H], wq, preferred_element_type=jnp.float32)
            yo = yo + jnp.dot(a[:, H:], wq, preferred_element_type=jnp.float32)
        ge = lnact(ye, g_ref[...], be_ref[...])
        go = lnact(yo, g_ref[...], be_ref[...])
        inter = jnp.stack([ge, go], axis=1).reshape(BLK, D)
        out_ref[0] = inter + node_ref[0]

    agg4 = agg.reshape(B, NQ, N // 2, 2 * H)
    return pl.pallas_call(
        body,
        grid=(B, N // BLK),
        in_specs=[
            pl.BlockSpec((1, NQ, PB, 2 * H), lambda bi, ni: (bi, 0, ni, 0)),
            pl.BlockSpec((1, BLK, D), lambda bi, ni: (bi, ni, 0)),
            pl.BlockSpec((D, D), lambda bi, ni: (0, 0)),
            pl.BlockSpec((1, D), lambda bi, ni: (0, 0)),
            pl.BlockSpec((1, D), lambda bi, ni: (0, 0)),
            pl.BlockSpec((1, D), lambda bi, ni: (0, 0)),
        ],
        out_specs=pl.BlockSpec((1, BLK, D), lambda bi, ni: (bi, ni, 0)),
        out_shape=jax.ShapeDtypeStruct((B, N, D), jnp.float32),
    )(agg4, node, W, bvec.reshape(1, D), gamma.reshape(1, D), beta.reshape(1, D))


def kernel(node_embeddings, edges, W, b, gamma, beta):
    B, N, D = node_embeddings.shape
    E = edges.shape[0]
    NQ = 4
    H = D // NQ

    tbl = node_embeddings.reshape(B, N * NQ, H)

    EPT = E // NT
    NBLK = (-(-EPT // KB) + 3) & ~3
    padt = NBLK * KB - EPT
    src = edges[:, 0].reshape(NT, EPT)
    dst = edges[:, 1].reshape(NT, EPT)
    if padt:
        pidx = jnp.arange(padt, dtype=jnp.int32)
        src = jnp.concatenate(
            [src, jnp.broadcast_to(pidx % N, (NT, padt))], axis=1)
        dst = jnp.concatenate(
            [dst, jnp.broadcast_to(N + pidx % PAD_ROWS, (NT, padt))], axis=1)
    src_r = src.reshape(NT, NBLK, KB)
    dst_r = dst.reshape(NT, NBLK, KB)

    agg = _sc_aggregate(tbl, src_r, dst_r, B, N, H, NQ, NBLK)

    BLK = 2000
    return _tc_epilogue(agg, node_embeddings, W, b, gamma, beta, B, N, D, H, NQ, BLK)


# speedup vs baseline: 1.0340x; 1.0340x over previous
import functools

import jax
import jax.numpy as jnp
import numpy as np
from jax import lax
from jax.experimental import pallas as pl
from jax.experimental.pallas import tpu as pltpu
from jax.experimental.pallas import tpu_sc as plsc

NT = 16
NC = 2
KB = 128
PAD_ROWS = 240


def _sc_aggregate(tbl, src_r, dst_r, B, N, H, NQ, NBLK):
    NP = N + PAD_ROWS
    ZR = NP // NT
    ZSUB = ZR // 2
    WS = (N // NT) & ~7
    WTAIL = N - NT * WS

    mesh = plsc.VectorSubcoreMesh(
        core_axis_name="c", subcore_axis_name="s",
        num_cores=NC, num_subcores=NT)

    @functools.partial(
        pl.kernel,
        out_type=jax.ShapeDtypeStruct((B * NQ, N, H), jnp.float32),
        mesh=mesh,
        scratch_types=[
            pltpu.VMEM((NBLK, KB), jnp.int32),
            pltpu.VMEM((NBLK, KB), jnp.int32),
            pltpu.VMEM((NBLK, KB), jnp.int32),
            pltpu.VMEM((4, KB, H), jnp.float32),
            pltpu.VMEM((ZSUB, H), jnp.float32),
            pltpu.VMEM_SHARED((NP, H), jnp.float32),
            [pltpu.SemaphoreType.DMA] * 4,
        ],
        compiler_params=pltpu.CompilerParams(use_tc_tiling_on_sc=False),
    )
    def agg_kernel(tbl_hbm, src_hbm, dst_hbm, out_hbm,
                   src_v, idx_v, dst_v, rows_v, zbuf_v, acc_sh, gsems):
        c = lax.axis_index("c")
        s = lax.axis_index("s")

        pltpu.sync_copy(src_hbm.at[s], src_v)
        pltpu.sync_copy(dst_hbm.at[s], dst_v)

        def fill_idx(q):
            def frow(i, _):
                for jj in range(KB // 16):
                    sl = pl.ds(jj * 16, 16)
                    idx_v[i, sl] = src_v[i, sl] * NQ + q
                return 0
            lax.fori_loop(0, NBLK, frow, 0)

        zvec = jnp.zeros((16,), jnp.float32)

        def zrow(i, _):
            for j in range(H // 16):
                zbuf_v[i, pl.ds(j * 16, 16)] = zvec
            return 0

        lax.fori_loop(0, ZSUB, zrow, 0)

        def chunk(b, q):
            tblk = tbl_hbm.at[b]
            outk = out_hbm.at[b * NQ + q]
            for r in range(2):
                pltpu.sync_copy(zbuf_v, acc_sh.at[pl.ds(s * ZR + r * ZSUB, ZSUB)])
            plsc.subcore_barrier()

            def issue_g(j, t):
                pltpu.async_copy(tblk.at[idx_v.at[j]], rows_v.at[t], gsems[t])

            def wait_g(j, t):
                pltpu.make_async_copy(
                    tblk.at[idx_v.at[j]], rows_v.at[t], gsems[t]).wait()

            def scatter(j, t):
                pltpu.sync_copy(rows_v.at[t], acc_sh.at[dst_v.at[j]], add=True)

            for t in range(4):
                issue_g(t, t)

            def step(g, _):
                j0 = 4 * g
                for t in range(4):
                    wait_g(j0 + t, t)
                    scatter(j0 + t, t)
                    issue_g(j0 + t + 4, t)
                return 0

            lax.fori_loop(0, NBLK // 4 - 1, step, 0)
            j0 = NBLK - 4
            for t in range(4):
                wait_g(j0 + t, t)
                scatter(j0 + t, t)
            plsc.subcore_barrier()

            pltpu.sync_copy(acc_sh.at[pl.ds(s * WS, WS)],
                            outk.at[pl.ds(s * WS, WS)])
            if WTAIL:
                @pl.when(s == NT - 1)
                def _():
                    pltpu.sync_copy(acc_sh.at[pl.ds(NT * WS, WTAIL)],
                                    outk.at[pl.ds(NT * WS, WTAIL)])
            plsc.subcore_barrier()

        qpc = NQ // NC
        for half in range(NC):
            @pl.when(c == half)
            def _():
                for j in range(qpc):
                    q = half * qpc + j
                    fill_idx(q)
                    for b in range(B):
                        chunk(b, q)

    return agg_kernel(tbl, src_r, dst_r)


def _tc_epilogue(agg, node, W, bvec, gamma, beta, bidx, Bfull, carry,
                 N, D, H, NQ, BLK):
    PB = BLK // 2

    def lnact(y, g, be):
        mu = jnp.mean(y, axis=-1, keepdims=True)
        yc = y - mu
        var = jnp.mean(yc * yc, axis=-1, keepdims=True)
        ln = yc * lax.rsqrt(var + 1e-5) * g + be
        return 0.5 * ln * (1.0 + lax.erf(ln * np.float32(1.0 / np.sqrt(2.0))))

    def body(agg_ref, node_ref, w_ref, b_ref, g_ref, be_ref, out_ref):
        ye = b_ref[...]
        yo = b_ref[...]
        for q in range(NQ):
            a = agg_ref[0, q]
            wq = w_ref[q * H:(q + 1) * H, :]
            ye = ye + jnp.dot(a[:, :H], wq, preferred_element_type=jnp.float32)
            yo = yo + jnp.dot(a[:, H:], wq, preferred_element_type=jnp.float32)
        ge = lnact(ye, g_ref[...], be_ref[...])
        go = lnact(yo, g_ref[...], be_ref[...])
        inter = jnp.stack([ge, go], axis=1).reshape(BLK, D)
        out_ref[0] = inter + node_ref[0]

    agg4 = agg.reshape(1, NQ, N // 2, 2 * H)
    in_specs = [
        pl.BlockSpec((1, NQ, PB, 2 * H), lambda ni: (0, 0, ni, 0)),
        pl.BlockSpec((1, BLK, D), lambda ni: (0, ni, 0)),
        pl.BlockSpec((D, D), lambda ni: (0, 0)),
        pl.BlockSpec((1, D), lambda ni: (0, 0)),
        pl.BlockSpec((1, D), lambda ni: (0, 0)),
        pl.BlockSpec((1, D), lambda ni: (0, 0)),
    ]
    args = [agg4, node, W, bvec.reshape(1, D),
            gamma.reshape(1, D), beta.reshape(1, D)]
    kwargs = {}
    if carry is not None:
        def body2(agg_ref, node_ref, w_ref, b_ref, g_ref, be_ref,
                  carry_ref, out_ref):
            body(agg_ref, node_ref, w_ref, b_ref, g_ref, be_ref, out_ref)
        in_specs.append(pl.BlockSpec(memory_space=pl.ANY))
        args.append(carry)
        kwargs["input_output_aliases"] = {6: 0}
        fn = body2
    else:
        fn = body
    return pl.pallas_call(
        fn,
        grid=(N // BLK,),
        in_specs=in_specs,
        out_specs=pl.BlockSpec((1, BLK, D), lambda ni: (bidx, ni, 0)),
        out_shape=jax.ShapeDtypeStruct((Bfull, N, D), jnp.float32),
        **kwargs,
    )(*args)


def kernel(node_embeddings, edges, W, b, gamma, beta):
    B, N, D = node_embeddings.shape
    E = edges.shape[0]
    NQ = 4
    H = D // NQ

    tbl = node_embeddings.reshape(B, N * NQ, H)

    EPT = E // NT
    NBLK = (-(-EPT // KB) + 3) & ~3
    padt = NBLK * KB - EPT
    src = edges[:, 0].reshape(NT, EPT)
    dst = edges[:, 1].reshape(NT, EPT)
    if padt:
        pidx = jnp.arange(padt, dtype=jnp.int32)
        src = jnp.concatenate(
            [src, jnp.broadcast_to(pidx % N, (NT, padt))], axis=1)
        dst = jnp.concatenate(
            [dst, jnp.broadcast_to(N + pidx % PAD_ROWS, (NT, padt))], axis=1)
    src_r = src.reshape(NT, NBLK, KB)
    dst_r = dst.reshape(NT, NBLK, KB)

    BLK = 2000
    out = None
    for bi in range(B):
        agg_b = _sc_aggregate(tbl[bi:bi + 1], src_r, dst_r, 1, N, H, NQ, NBLK)
        out = _tc_epilogue(agg_b.reshape(NQ, N, H),
                           node_embeddings[bi:bi + 1], W, b, gamma, beta,
                           bi, B, out, N, D, H, NQ, BLK)
    return out


# speedup vs baseline: 1.1367x; 1.0993x over previous
import functools

import jax
import jax.numpy as jnp
import numpy as np
from jax import lax
from jax.experimental import pallas as pl
from jax.experimental.pallas import tpu as pltpu
from jax.experimental.pallas import tpu_sc as plsc

NT = 16
NC = 2
KB = 128
PAD_ROWS = 240


def _sc_aggregate(tbl, src_r, dst_r, B, N, H, NQ, NBLK):
    NP = N + PAD_ROWS
    ZR = NP // NT
    ZSUB = ZR // 2
    WS = (N // NT) & ~7
    WTAIL = N - NT * WS

    mesh = plsc.VectorSubcoreMesh(
        core_axis_name="c", subcore_axis_name="s",
        num_cores=NC, num_subcores=NT)

    @functools.partial(
        pl.kernel,
        out_type=jax.ShapeDtypeStruct((B * NQ, N, H), jnp.float32),
        mesh=mesh,
        scratch_types=[
            pltpu.VMEM((NBLK, KB), jnp.int32),
            pltpu.VMEM((NBLK, KB), jnp.int32),
            pltpu.VMEM((NBLK, KB), jnp.int32),
            pltpu.VMEM((4, KB, H), jnp.float32),
            pltpu.VMEM((ZSUB, H), jnp.float32),
            pltpu.VMEM_SHARED((NP, H), jnp.float32),
            [pltpu.SemaphoreType.DMA] * 4,
        ],
        compiler_params=pltpu.CompilerParams(use_tc_tiling_on_sc=False),
    )
    def agg_kernel(tbl_hbm, src_hbm, dst_hbm, out_hbm,
                   src_v, idx_v, dst_v, rows_v, zbuf_v, acc_sh, gsems):
        c = lax.axis_index("c")
        s = lax.axis_index("s")

        pltpu.sync_copy(src_hbm.at[s], src_v)
        pltpu.sync_copy(dst_hbm.at[s], dst_v)

        def fill_idx(q):
            def frow(i, _):
                for jj in range(KB // 16):
                    sl = pl.ds(jj * 16, 16)
                    idx_v[i, sl] = src_v[i, sl] * NQ + q
                return 0
            lax.fori_loop(0, NBLK, frow, 0)

        zvec = jnp.zeros((16,), jnp.float32)

        def zrow(i, _):
            for j in range(H // 16):
                zbuf_v[i, pl.ds(j * 16, 16)] = zvec
            return 0

        lax.fori_loop(0, ZSUB, zrow, 0)

        def chunk(b, q):
            tblk = tbl_hbm.at[b]
            outk = out_hbm.at[b * NQ + q]
            for r in range(2):
                pltpu.sync_copy(zbuf_v, acc_sh.at[pl.ds(s * ZR + r * ZSUB, ZSUB)])
            plsc.subcore_barrier()

            def issue_g(j, t):
                pltpu.async_copy(tblk.at[idx_v.at[j]], rows_v.at[t], gsems[t])

            def wait_g(j, t):
                pltpu.make_async_copy(
                    tblk.at[idx_v.at[j]], rows_v.at[t], gsems[t]).wait()

            def scatter(j, t):
                pltpu.sync_copy(rows_v.at[t], acc_sh.at[dst_v.at[j]], add=True)

            for t in range(4):
                issue_g(t, t)

            def step(g, _):
                j0 = 4 * g
                for t in range(4):
                    wait_g(j0 + t, t)
                    scatter(j0 + t, t)
                    issue_g(j0 + t + 4, t)
                return 0

            lax.fori_loop(0, NBLK // 4 - 1, step, 0)
            j0 = NBLK - 4
            for t in range(4):
                wait_g(j0 + t, t)
                scatter(j0 + t, t)
            plsc.subcore_barrier()

            pltpu.sync_copy(acc_sh.at[pl.ds(s * WS, WS)],
                            outk.at[pl.ds(s * WS, WS)])
            if WTAIL:
                @pl.when(s == NT - 1)
                def _():
                    pltpu.sync_copy(acc_sh.at[pl.ds(NT * WS, WTAIL)],
                                    outk.at[pl.ds(NT * WS, WTAIL)])
            plsc.subcore_barrier()

        qpc = NQ // NC
        for half in range(NC):
            @pl.when(c == half)
            def _():
                for j in range(qpc):
                    q = half * qpc + j
                    fill_idx(q)
                    for b in range(B):
                        chunk(b, q)

    return agg_kernel(tbl, src_r, dst_r)


def _tc_epilogue(agg, node, W, bvec, gamma, beta, B, N, D, H, NQ, BLK):
    PB = BLK // 2

    def lnact(y, g, be):
        mu = jnp.mean(y, axis=-1, keepdims=True)
        yc = y - mu
        var = jnp.mean(yc * yc, axis=-1, keepdims=True)
        ln = yc * lax.rsqrt(var + 1e-5) * g + be
        return 0.5 * ln * (1.0 + lax.erf(ln * np.float32(1.0 / np.sqrt(2.0))))

    def body(agg_ref, node_ref, w_ref, b_ref, g_ref, be_ref, out_ref):
        ye = b_ref[...]
        yo = b_ref[...]
        for q in range(NQ):
            a = agg_ref[0, q]
            wq = w_ref[q * H:(q + 1) * H, :]
            ye = ye + jnp.dot(a[:, :H], wq, preferred_element_type=jnp.float32)
            yo = yo + jnp.dot(a[:, H:], wq, preferred_element_type=jnp.float32)
        ge = lnact(ye, g_ref[...], be_ref[...])
        go = lnact(yo, g_ref[...], be_ref[...])
        inter = jnp.stack([ge, go], axis=1).reshape(BLK, D)
        out_ref[0] = inter + node_ref[0]

    agg4 = agg.reshape(B, NQ, N // 2, 2 * H)
    return pl.pallas_call(
        body,
        grid=(B, N // BLK),
        in_specs=[
            pl.BlockSpec((1, NQ, PB, 2 * H), lambda bi, ni: (bi, 0, ni, 0)),
            pl.BlockSpec((1, BLK, D), lambda bi, ni: (bi, ni, 0)),
            pl.BlockSpec((D, D), lambda bi, ni: (0, 0)),
            pl.BlockSpec((1, D), lambda bi, ni: (0, 0)),
            pl.BlockSpec((1, D), lambda bi, ni: (0, 0)),
            pl.BlockSpec((1, D), lambda bi, ni: (0, 0)),
        ],
        out_specs=pl.BlockSpec((1, BLK, D), lambda bi, ni: (bi, ni, 0)),
        out_shape=jax.ShapeDtypeStruct((B, N, D), jnp.float32),
    )(agg4, node, W, bvec.reshape(1, D), gamma.reshape(1, D), beta.reshape(1, D))


def kernel(node_embeddings, edges, W, b, gamma, beta):
    B, N, D = node_embeddings.shape
    E = edges.shape[0]
    NQ = 4
    H = D // NQ

    tbl = node_embeddings.reshape(B, N * NQ, H)

    EPT = E // NT
    NBLK = (-(-EPT // KB) + 3) & ~3
    padt = NBLK * KB - EPT
    src = edges[:, 0].reshape(NT, EPT)
    dst = edges[:, 1].reshape(NT, EPT)
    if padt:
        pidx = jnp.arange(padt, dtype=jnp.int32)
        src = jnp.concatenate(
            [src, jnp.broadcast_to(pidx % N, (NT, padt))], axis=1)
        dst = jnp.concatenate(
            [dst, jnp.broadcast_to(N + pidx % PAD_ROWS, (NT, padt))], axis=1)
    src_r = src.reshape(NT, NBLK, KB)
    dst_r = dst.reshape(NT, NBLK, KB)

    BLK = 2000
    agg = _sc_aggregate(tbl, src_r, dst_r, B, N, H, NQ, NBLK)
    return _tc_epilogue(agg, node_embeddings, W, b, gamma, beta,
                        B, N, D, H, NQ, BLK)
